# baseline (device time: 9556 ns/iter reference)
import jax
import jax.numpy as jnp
from jax import lax
from jax.experimental import pallas as pl
from jax.experimental.pallas import tpu as pltpu

N_DEV = 4


def kernel(x):
    m, n = x.shape

    def body(
        x_hbm,
        out_hbm,
        x_vmem,
        send_ref,
        comm_ref,
        out_vmem,
        send_sems,
        recv_sems,
        local_sems,
    ):
        my = lax.axis_index("i")
        peers = (my ^ 1, 3 - my, my ^ 2)

        in_dma = pltpu.make_async_copy(x_hbm, x_vmem, local_sems.at[0])
        in_dma.start()

        barrier_sem = pltpu.get_barrier_semaphore()
        for nbr in peers:
            pl.semaphore_signal(
                barrier_sem, inc=1,
                device_id=(nbr,), device_id_type=pl.DeviceIdType.MESH,
            )
        in_dma.wait()
        send_ref[...] = x_vmem[...].astype(jnp.bfloat16)
        pl.semaphore_wait(barrier_sem, 3)

        rdmas = []
        for k, nbr in enumerate(peers):
            rdma = pltpu.make_async_remote_copy(
                src_ref=send_ref,
                dst_ref=comm_ref.at[k],
                send_sem=send_sems.at[k],
                recv_sem=recv_sems.at[k],
                device_id=(nbr,),
                device_id_type=pl.DeviceIdType.MESH,
            )
            rdma.start()
            rdmas.append(rdma)

        rdmas[0].wait_recv()
        acc = x_vmem[...] + comm_ref[0].astype(jnp.float32)
        rdmas[1].wait_recv()
        acc = acc + comm_ref[1].astype(jnp.float32)
        rdmas[2].wait_recv()
        out_vmem[...] = (acc + comm_ref[2].astype(jnp.float32)).astype(
            jnp.bfloat16
        )

        out_dma = pltpu.make_async_copy(out_vmem, out_hbm, local_sems.at[1])
        out_dma.start()
        for rdma in rdmas:
            rdma.wait_send()
        out_dma.wait()

    return pl.pallas_call(
        body,
        out_shape=jax.ShapeDtypeStruct((m, n), jnp.bfloat16),
        in_specs=[pl.BlockSpec(memory_space=pl.ANY)],
        out_specs=pl.BlockSpec(memory_space=pl.ANY),
        scratch_shapes=[
            pltpu.VMEM((m, n), jnp.float32),
            pltpu.VMEM((m, n), jnp.bfloat16),
            pltpu.VMEM((3, m, n), jnp.bfloat16),
            pltpu.VMEM((m, n), jnp.bfloat16),
            pltpu.SemaphoreType.DMA((3,)),
            pltpu.SemaphoreType.DMA((3,)),
            pltpu.SemaphoreType.DMA((2,)),
        ],
        compiler_params=pltpu.CompilerParams(collective_id=0),
    )(x)


# device time: 9402 ns/iter; 1.0164x vs baseline; 1.0164x over previous
import jax
import jax.numpy as jnp
from jax import lax
from jax.experimental import pallas as pl
from jax.experimental.pallas import tpu as pltpu

N_DEV = 4
N_CHUNK = 2


def kernel(x):
    m, n = x.shape
    mc = m // N_CHUNK

    def body(x_ref, out_ref, send_ref, comm_ref, send_sems, recv_sems):
        my = lax.axis_index("i")
        peers = (my ^ 1, 3 - my, my ^ 2)

        barrier_sem = pltpu.get_barrier_semaphore()
        for nbr in peers:
            pl.semaphore_signal(
                barrier_sem, inc=1,
                device_id=(nbr,), device_id_type=pl.DeviceIdType.MESH,
            )

        send_ref[...] = x_ref[...].astype(jnp.bfloat16)
        pl.semaphore_wait(barrier_sem, 3)

        rdmas = []
        for c in range(N_CHUNK):
            rows = pl.ds(c * mc, mc)
            for k, nbr in enumerate(peers):
                rdma = pltpu.make_async_remote_copy(
                    src_ref=send_ref.at[rows],
                    dst_ref=comm_ref.at[k, rows],
                    send_sem=send_sems.at[c, k],
                    recv_sem=recv_sems.at[c, k],
                    device_id=(nbr,),
                    device_id_type=pl.DeviceIdType.MESH,
                )
                rdma.start()
                rdmas.append(rdma)

        for c in range(N_CHUNK):
            rows = pl.ds(c * mc, mc)
            for k in range(3):
                rdmas[c * 3 + k].wait_recv()
            acc = (
                x_ref[rows, :] + comm_ref[0, rows, :].astype(jnp.float32)
            ) + (
                comm_ref[1, rows, :].astype(jnp.float32)
                + comm_ref[2, rows, :].astype(jnp.float32)
            )
            out_ref[rows, :] = acc.astype(jnp.bfloat16)

        for rdma in rdmas:
            rdma.wait_send()

    return pl.pallas_call(
        body,
        out_shape=jax.ShapeDtypeStruct((m, n), jnp.bfloat16),
        in_specs=[pl.BlockSpec(memory_space=pltpu.VMEM)],
        out_specs=pl.BlockSpec(memory_space=pltpu.VMEM),
        scratch_shapes=[
            pltpu.VMEM((m, n), jnp.bfloat16),
            pltpu.VMEM((3, m, n), jnp.bfloat16),
            pltpu.SemaphoreType.DMA((N_CHUNK, 3)),
            pltpu.SemaphoreType.DMA((N_CHUNK, 3)),
        ],
        compiler_params=pltpu.CompilerParams(collective_id=0),
    )(x)


# device time: 9361 ns/iter; 1.0208x vs baseline; 1.0044x over previous
import jax
import jax.numpy as jnp
from jax import lax
from jax.experimental import pallas as pl
from jax.experimental.pallas import tpu as pltpu

N_DEV = 4


def kernel(x):
    m, n = x.shape

    def body(x_ref, out_ref, send_ref, comm_ref, send_sems, recv_sems):
        my = lax.axis_index("i")
        peers = (my ^ 1, 3 - my, my ^ 2)

        barrier_sem = pltpu.get_barrier_semaphore()
        for nbr in peers:
            pl.semaphore_signal(
                barrier_sem, inc=1,
                device_id=(nbr,), device_id_type=pl.DeviceIdType.MESH,
            )

        send_ref[...] = x_ref[...].astype(jnp.bfloat16)
        pl.semaphore_wait(barrier_sem, 3)

        rdmas = []
        for k, nbr in enumerate(peers):
            rdma = pltpu.make_async_remote_copy(
                src_ref=send_ref,
                dst_ref=comm_ref.at[k],
                send_sem=send_sems.at[k],
                recv_sem=recv_sems.at[k],
                device_id=(nbr,),
                device_id_type=pl.DeviceIdType.MESH,
            )
            rdma.start()
            rdmas.append(rdma)

        rdmas[0].wait_recv()
        acc = x_ref[...] + comm_ref[0].astype(jnp.float32)
        rdmas[1].wait_recv()
        acc = acc + comm_ref[1].astype(jnp.float32)
        rdmas[2].wait_recv()
        out_ref[...] = (acc + comm_ref[2].astype(jnp.float32)).astype(
            jnp.bfloat16
        )

        for rdma in rdmas:
            rdma.wait_send()

    return pl.pallas_call(
        body,
        out_shape=jax.ShapeDtypeStruct((m, n), jnp.bfloat16),
        in_specs=[pl.BlockSpec(memory_space=pltpu.VMEM)],
        out_specs=pl.BlockSpec(memory_space=pltpu.VMEM),
        scratch_shapes=[
            pltpu.VMEM((m, n), jnp.bfloat16),
            pltpu.VMEM((3, m, n), jnp.bfloat16),
            pltpu.SemaphoreType.DMA((3,)),
            pltpu.SemaphoreType.DMA((3,)),
        ],
        compiler_params=pltpu.CompilerParams(collective_id=0),
    )(x)
